# baseline (speedup 1.0000x reference)
import jax
import jax.numpy as jnp
from jax import lax
from jax.experimental import pallas as pl
from jax.experimental.pallas import tpu as pltpu
from jax.experimental.pallas import tpu_sc as plsc

N_FEAT = 1000000
N_DIM = 16
BATCH = 16384

_info = plsc.get_sparse_core_info()
_NC, _NS, _NL = _info.num_cores, _info.num_subcores, _info.num_lanes
_NW = _NC * _NS
_BPW = BATCH // _NW
_HG = 8
_NH = _BPW // _HG
_SETW = _HG * 128


def _sc_body(idx_hbm, mu_hbm, lv_hbm, bias_hbm, eps_hbm,
             bias_out, vec_out,
             idx_v, idx_pad, eps_v, vec_v, mu_buf, lv_buf, bias_v,
             sem_b, sem_mu0, sem_mu1, sem_mu2, sem_lv0, sem_lv1, sem_lv2):
    sems_mu = (sem_mu0, sem_mu1, sem_mu2)
    sems_lv = (sem_lv0, sem_lv1, sem_lv2)
    wid = lax.axis_index("s") * _NC + lax.axis_index("c")
    base = pl.multiple_of(wid * _BPW, _BPW)
    pltpu.sync_copy(idx_hbm.at[pl.ds(base, _BPW)], idx_v)
    pltpu.sync_copy(idx_hbm.at[pl.ds(base, _BPW)], idx_pad.at[pl.ds(0, _BPW)])
    idx_pad[pl.ds(_BPW, _NL)] = jnp.zeros((_NL,), jnp.int32)
    cp_b = pltpu.async_copy(bias_hbm.at[idx_v], bias_v, sem_b)
    pltpu.sync_copy(eps_hbm.at[:, pl.ds(base, _BPW)], eps_v)

    iota = lax.iota(jnp.int32, _NL)

    def issue(h, p):
        off = p * _SETW
        ivec = idx_pad[pl.ds(h * _HG, _NL)]
        for j in range(_HG):
            col = ivec[j]
            col_al = pl.multiple_of((col // 128) * 128, 128)
            pltpu.async_copy(mu_hbm.at[:, pl.ds(col_al, 128)],
                             mu_buf.at[:, pl.ds(off + j * 128, 128)],
                             sems_mu[p])
            pltpu.async_copy(lv_hbm.at[:, pl.ds(col_al, 128)],
                             lv_buf.at[:, pl.ds(off + j * 128, 128)],
                             sems_lv[p])

    def extract(h, p):
        off = p * _SETW
        pltpu.make_async_copy(mu_hbm.at[:, pl.ds(0, _SETW)],
                              mu_buf.at[:, pl.ds(off, _SETW)],
                              sems_mu[p]).wait()
        pltpu.make_async_copy(lv_hbm.at[:, pl.ds(0, _SETW)],
                              lv_buf.at[:, pl.ds(off, _SETW)],
                              sems_lv[p]).wait()
        ivec = idx_pad[pl.ds(h * _HG, _NL)]
        for j in range(_HG):
            pos_s = h * _HG + j
            col = ivec[j]
            rem = col - (col // 128) * 128
            cidx = jnp.full((_NL,), off + j * 128, jnp.int32) + rem
            mu_col = plsc.load_gather(mu_buf, [iota, cidx])
            lv_col = plsc.load_gather(lv_buf, [iota, cidx])
            pos = jnp.full((_NL,), pos_s, jnp.int32)
            eps_col = plsc.load_gather(eps_v, [iota, pos])
            val = eps_col * jnp.exp(lv_col * 0.5) + mu_col
            plsc.store_scatter(vec_v, [iota, pos], val)

    def step(u, carry):
        for p in range(3):
            h = u * 3 + p

            @pl.when(h < _NH)
            def _():
                issue(h, p)

            he = h - 2
            pe = (p + 1) % 3

            @pl.when(jnp.logical_and(he >= 0, he < _NH))
            def _():
                extract(he, pe)
        return carry

    lax.fori_loop(0, (_NH + 2 + 2) // 3, step, 0)
    pltpu.sync_copy(vec_v, vec_out.at[:, pl.ds(base, _BPW)])
    cp_b.wait()
    pltpu.sync_copy(bias_v, bias_out.at[pl.ds(base, _BPW)])


def kernel(index, mu_w, lv_w, bias_w, eps):
    idx = index.astype(jnp.int32)
    f = pl.kernel(
        _sc_body,
        out_type=(
            jax.ShapeDtypeStruct((BATCH,), jnp.float32),
            jax.ShapeDtypeStruct((N_DIM, BATCH), jnp.float32),
        ),
        mesh=plsc.VectorSubcoreMesh(core_axis_name="c", subcore_axis_name="s"),
        compiler_params=pltpu.CompilerParams(
            needs_layout_passes=False, skip_device_barrier=True),
        scratch_types=[
            pltpu.VMEM((_BPW,), jnp.int32),
            pltpu.VMEM((_BPW + _NL,), jnp.int32),
            pltpu.VMEM((N_DIM, _BPW), jnp.float32),
            pltpu.VMEM((N_DIM, _BPW), jnp.float32),
            pltpu.VMEM((N_DIM, 3 * _SETW), jnp.float32),
            pltpu.VMEM((N_DIM, 3 * _SETW), jnp.float32),
            pltpu.VMEM((_BPW,), jnp.float32),
            pltpu.SemaphoreType.DMA,
            pltpu.SemaphoreType.DMA,
            pltpu.SemaphoreType.DMA,
            pltpu.SemaphoreType.DMA,
            pltpu.SemaphoreType.DMA,
            pltpu.SemaphoreType.DMA,
            pltpu.SemaphoreType.DMA,
        ],
    )
    bias, vec_t = f(idx, mu_w.T, lv_w.T, bias_w.reshape(N_FEAT), eps.T)
    return bias, vec_t.T


# speedup vs baseline: 1.0086x; 1.0086x over previous
import jax
import jax.numpy as jnp
from jax import lax
from jax.experimental import pallas as pl
from jax.experimental.pallas import tpu as pltpu
from jax.experimental.pallas import tpu_sc as plsc

N_FEAT = 1000000
N_DIM = 16
BATCH = 16384

_info = plsc.get_sparse_core_info()
_NC, _NS, _NL = _info.num_cores, _info.num_subcores, _info.num_lanes
_NW = _NC * _NS
_BPW = BATCH // _NW
_HG = 4
_NH = _BPW // _HG
_NSET = 6
_LAG = 4
_SETW = _HG * 128


def _sc_body(idx_hbm, mu_hbm, lv_hbm, bias_hbm, eps_hbm,
             bias_out, vec_out,
             idx_v, idx_pad, eps_v, vec_v, mu_buf, lv_buf, bias_v,
             sem_b, sem_mu0, sem_mu1, sem_mu2, sem_mu3, sem_mu4, sem_mu5,
             sem_lv0, sem_lv1, sem_lv2, sem_lv3, sem_lv4, sem_lv5):
    sems_mu = (sem_mu0, sem_mu1, sem_mu2, sem_mu3, sem_mu4, sem_mu5)
    sems_lv = (sem_lv0, sem_lv1, sem_lv2, sem_lv3, sem_lv4, sem_lv5)
    wid = lax.axis_index("s") * _NC + lax.axis_index("c")
    base = pl.multiple_of(wid * _BPW, _BPW)
    pltpu.sync_copy(idx_hbm.at[pl.ds(base, _BPW)], idx_v)
    pltpu.sync_copy(idx_hbm.at[pl.ds(base, _BPW)], idx_pad.at[pl.ds(0, _BPW)])
    idx_pad[pl.ds(_BPW, _NL)] = jnp.zeros((_NL,), jnp.int32)
    cp_b = pltpu.async_copy(bias_hbm.at[idx_v], bias_v, sem_b)
    pltpu.sync_copy(eps_hbm.at[:, pl.ds(base, _BPW)], eps_v)

    iota = lax.iota(jnp.int32, _NL)

    def issue(h, p):
        off = p * _SETW
        ivec = idx_pad[pl.ds(h * _HG, _NL)]
        for j in range(_HG):
            col = ivec[j]
            col_al = pl.multiple_of((col // 128) * 128, 128)
            pltpu.async_copy(mu_hbm.at[:, pl.ds(col_al, 128)],
                             mu_buf.at[:, pl.ds(off + j * 128, 128)],
                             sems_mu[p])
            pltpu.async_copy(lv_hbm.at[:, pl.ds(col_al, 128)],
                             lv_buf.at[:, pl.ds(off + j * 128, 128)],
                             sems_lv[p])

    def extract(h, p):
        off = p * _SETW
        pltpu.make_async_copy(mu_hbm.at[:, pl.ds(0, _SETW)],
                              mu_buf.at[:, pl.ds(off, _SETW)],
                              sems_mu[p]).wait()
        pltpu.make_async_copy(lv_hbm.at[:, pl.ds(0, _SETW)],
                              lv_buf.at[:, pl.ds(off, _SETW)],
                              sems_lv[p]).wait()
        ivec = idx_pad[pl.ds(h * _HG, _NL)]
        for j in range(_HG):
            pos_s = h * _HG + j
            col = ivec[j]
            rem = col - (col // 128) * 128
            cidx = jnp.full((_NL,), off + j * 128, jnp.int32) + rem
            mu_col = plsc.load_gather(mu_buf, [iota, cidx])
            lv_col = plsc.load_gather(lv_buf, [iota, cidx])
            pos = jnp.full((_NL,), pos_s, jnp.int32)
            eps_col = plsc.load_gather(eps_v, [iota, pos])
            val = eps_col * jnp.exp(lv_col * 0.5) + mu_col
            plsc.store_scatter(vec_v, [iota, pos], val)

    def step(u, carry):
        for p in range(_NSET):
            h = u * _NSET + p

            @pl.when(h < _NH)
            def _():
                issue(h, p)

            he = h - _LAG
            pe = (p + _NSET - _LAG) % _NSET

            @pl.when(jnp.logical_and(he >= 0, he < _NH))
            def _():
                extract(he, pe)
        return carry

    lax.fori_loop(0, (_NH + _LAG + _NSET - 1) // _NSET, step, 0)
    pltpu.sync_copy(vec_v, vec_out.at[:, pl.ds(base, _BPW)])
    cp_b.wait()
    pltpu.sync_copy(bias_v, bias_out.at[pl.ds(base, _BPW)])


def kernel(index, mu_w, lv_w, bias_w, eps):
    idx = index.astype(jnp.int32)
    f = pl.kernel(
        _sc_body,
        out_type=(
            jax.ShapeDtypeStruct((BATCH,), jnp.float32),
            jax.ShapeDtypeStruct((N_DIM, BATCH), jnp.float32),
        ),
        mesh=plsc.VectorSubcoreMesh(core_axis_name="c", subcore_axis_name="s"),
        compiler_params=pltpu.CompilerParams(
            needs_layout_passes=False, skip_device_barrier=True),
        scratch_types=[
            pltpu.VMEM((_BPW,), jnp.int32),
            pltpu.VMEM((_BPW + _NL,), jnp.int32),
            pltpu.VMEM((N_DIM, _BPW), jnp.float32),
            pltpu.VMEM((N_DIM, _BPW), jnp.float32),
            pltpu.VMEM((N_DIM, _NSET * _SETW), jnp.float32),
            pltpu.VMEM((N_DIM, _NSET * _SETW), jnp.float32),
            pltpu.VMEM((_BPW,), jnp.float32),
        ] + [pltpu.SemaphoreType.DMA] * 13,
    )
    bias, vec_t = f(idx, mu_w.T, lv_w.T, bias_w.reshape(N_FEAT), eps.T)
    return bias, vec_t.T
